# baseline (speedup 1.0000x reference)
import functools

import jax
import jax.numpy as jnp
from jax import lax
from jax.experimental import pallas as pl
from jax.experimental.pallas import tpu as pltpu
from jax.experimental.pallas import tpu_sc as plsc

_NC = 2
_NS = 16
_SW = 256
_G = 64


def _positional_lookup(table16, idx, d):
    n_rows = table16.shape[1] // _SW
    b = idx.shape[0]
    bpc = b // _NC
    n_chunks = bpc // _G
    mesh = plsc.VectorSubcoreMesh(core_axis_name="c", subcore_axis_name="s")

    @functools.partial(
        pl.kernel,
        mesh=mesh,
        out_type=jax.ShapeDtypeStruct((b, d), jnp.float32),
        scratch_types=[
            pltpu.VMEM((bpc,), jnp.int32),
            pltpu.VMEM((table16.shape[1],), jnp.float32),
            pltpu.VMEM((_G, _SW), jnp.float32),
            pltpu.VMEM((_G, _SW), jnp.float32),
            pltpu.SemaphoreType.DMA,
            pltpu.SemaphoreType.DMA,
        ],
    )
    def k(idx_hbm, tab_hbm, out_hbm, idx_v, tab_tile, buf0, buf1, w0, w1):
        bufs = (buf0, buf1)
        ws = (w0, w1)
        sid = lax.axis_index("s")
        cid = lax.axis_index("c")
        bbase = cid * bpc
        pltpu.sync_copy(idx_hbm.at[pl.ds(bbase, bpc)], idx_v)
        pltpu.sync_copy(tab_hbm.at[sid], tab_tile)
        col = sid * _SW

        def write(c, j):
            return pltpu.make_async_copy(
                bufs[j],
                out_hbm.at[pl.ds(bbase + c * _G, _G), pl.ds(col, _SW)],
                ws[j],
            )

        def assemble(c, j):
            buf = bufs[j]

            def vec_body(i, carry):
                vec = idx_v[pl.ds(c * _G + i * 16, 16)]
                for l in range(16):
                    s = vec[l]
                    r = i * 16 + l
                    for kk in range(_SW // 16):
                        buf[r, pl.ds(kk * 16, 16)] = tab_tile[
                            pl.ds(s * _SW + kk * 16, 16)
                        ]
                return carry

            lax.fori_loop(0, _G // 16, vec_body, 0)

        def body(i, carry):
            for j in range(2):
                c = 2 * i + j

                @pl.when(c >= 2)
                def _():
                    write(c - 2, j).wait()

                assemble(c, j)
                write(c, j).start()

            return carry

        lax.fori_loop(0, n_chunks // 2, body, 0)
        write(n_chunks - 2, 0).wait()
        write(n_chunks - 1, 1).wait()

    return k(idx, table16)


def kernel(x, weights):
    n_rows = weights.shape[0]
    d = weights.shape[1] * weights.shape[2]
    table = weights.reshape(n_rows, d)
    pad = (-n_rows) % 8
    if pad:
        table = jnp.pad(table, ((0, pad), (0, 0)))
    nr = table.shape[0]
    table16 = table.reshape(nr, _NS, _SW).transpose(1, 0, 2).reshape(_NS, nr * _SW)
    out = _positional_lookup(table16, x, d)
    return out


# speedup vs baseline: 1.0013x; 1.0013x over previous
import functools

import jax
import jax.numpy as jnp
from jax import lax
from jax.experimental import pallas as pl
from jax.experimental.pallas import tpu as pltpu
from jax.experimental.pallas import tpu_sc as plsc

_NC = 2
_NS = 16
_SW = 256
_G = 64


def _positional_lookup(table16, idx, d):
    n_rows = table16.shape[1]
    b = idx.shape[0]
    bpc = b // _NC
    n_chunks = bpc // _G
    mesh = plsc.VectorSubcoreMesh(core_axis_name="c", subcore_axis_name="s")

    @functools.partial(
        pl.kernel,
        mesh=mesh,
        out_type=jax.ShapeDtypeStruct((b, d), jnp.float32),
        scratch_types=[
            pltpu.VMEM((bpc,), jnp.int32),
            pltpu.VMEM((n_rows, _SW), jnp.float32),
            pltpu.VMEM((2, _G, _SW), jnp.float32),
            pltpu.SemaphoreType.DMA,
            pltpu.SemaphoreType.DMA,
            pltpu.SemaphoreType.DMA,
            pltpu.SemaphoreType.DMA,
        ],
    )
    def k(idx_hbm, tab_hbm, out_hbm, idx_v, tab_tile, bufs, g0, g1, w0, w1):
        gs = (g0, g1)
        ws = (w0, w1)
        sid = lax.axis_index("s")
        cid = lax.axis_index("c")
        bbase = cid * bpc
        pltpu.sync_copy(idx_hbm.at[pl.ds(bbase, bpc)], idx_v)
        pltpu.sync_copy(tab_hbm.at[sid], tab_tile)
        col = sid * _SW

        def assemble(c, j):
            def vec_body(i, carry):
                vec = idx_v[pl.ds(c * _G + i * 16, 16)]
                for l in range(16):
                    row_src = tab_tile.at[vec[l]]
                    row_dst = bufs.at[j, i * 16 + l]
                    for kk in range(_SW // 16):
                        row_dst[pl.ds(kk * 16, 16)] = row_src[pl.ds(kk * 16, 16)]
                return carry

            lax.fori_loop(0, _G // 16, vec_body, 0)

        def write(c, j):
            return pltpu.make_async_copy(
                bufs.at[j],
                out_hbm.at[pl.ds(bbase + c * _G, _G), pl.ds(col, _SW)],
                ws[j],
            )

        def body(i, carry):
            for j in range(2):
                c = 2 * i + j

                @pl.when(c >= 2)
                def _():
                    write(c - 2, j).wait()

                assemble(c, j)
                write(c, j).start()

            return carry

        lax.fori_loop(0, n_chunks // 2, body, 0)
        write(n_chunks - 2, 0).wait()
        write(n_chunks - 1, 1).wait()

    return k(idx, table16)


def kernel(x, weights):
    n_rows = weights.shape[0]
    d = weights.shape[1] * weights.shape[2]
    table = weights.reshape(n_rows, d)
    pad = (-n_rows) % 8
    if pad:
        table = jnp.pad(table, ((0, pad), (0, 0)))
    nr = table.shape[0]
    table16 = table.reshape(nr, _NS, _SW).transpose(1, 0, 2)
    out = _positional_lookup(table16, x, d)
    return out


# speedup vs baseline: 1.6466x; 1.6444x over previous
import functools

import jax
import jax.numpy as jnp
from jax import lax
from jax.experimental import pallas as pl
from jax.experimental.pallas import tpu as pltpu
from jax.experimental.pallas import tpu_sc as plsc

_NC = 2
_NS = 16
_SW = 256
_G = 64


def _positional_lookup(table16, idx, d):
    n_rows = table16.shape[1]
    b = idx.shape[0]
    bpc = b // _NC
    n_chunks = bpc // _G
    mesh = plsc.VectorSubcoreMesh(core_axis_name="c", subcore_axis_name="s")

    @functools.partial(
        pl.kernel,
        mesh=mesh,
        out_type=jax.ShapeDtypeStruct((b, d), jnp.float32),
        scratch_types=[
            pltpu.VMEM((bpc,), jnp.int32),
            pltpu.VMEM((n_rows, _SW), jnp.float32),
            pltpu.VMEM((2, _G, _SW), jnp.float32),
            pltpu.SemaphoreType.DMA,
            pltpu.SemaphoreType.DMA,
            pltpu.SemaphoreType.DMA,
            pltpu.SemaphoreType.DMA,
        ],
    )
    def k(idx_hbm, tab_hbm, out_hbm, idx_v, tab_tile, bufs, g0, g1, w0, w1):
        gs = (g0, g1)
        ws = (w0, w1)
        sid = lax.axis_index("s")
        cid = lax.axis_index("c")
        bbase = cid * bpc
        pltpu.sync_copy(idx_hbm.at[pl.ds(bbase, bpc)], idx_v)
        pltpu.sync_copy(tab_hbm.at[sid], tab_tile)
        col = sid * _SW

        def assemble(c, j):
            @plsc.parallel_loop(0, _G // 16, unroll=2)
            def vec_body(i):
                vec = idx_v[pl.ds(c * _G + i * 16, 16)]
                for l in range(16):
                    row_src = tab_tile.at[vec[l]]
                    row_dst = bufs.at[j, i * 16 + l]
                    vals = [
                        row_src[pl.ds(kk * 16, 16)] for kk in range(_SW // 16)
                    ]
                    for kk in range(_SW // 16):
                        row_dst[pl.ds(kk * 16, 16)] = vals[kk]

        def write(c, j):
            return pltpu.make_async_copy(
                bufs.at[j],
                out_hbm.at[pl.ds(bbase + c * _G, _G), pl.ds(col, _SW)],
                ws[j],
            )

        def body(i, carry):
            for j in range(2):
                c = 2 * i + j

                @pl.when(c >= 2)
                def _():
                    write(c - 2, j).wait()

                assemble(c, j)
                write(c, j).start()

            return carry

        lax.fori_loop(0, n_chunks // 2, body, 0)
        write(n_chunks - 2, 0).wait()
        write(n_chunks - 1, 1).wait()

    return k(idx, table16)


def kernel(x, weights):
    n_rows = weights.shape[0]
    d = weights.shape[1] * weights.shape[2]
    table = weights.reshape(n_rows, d)
    pad = (-n_rows) % 8
    if pad:
        table = jnp.pad(table, ((0, pad), (0, 0)))
    nr = table.shape[0]
    table16 = table.reshape(nr, _NS, _SW).transpose(1, 0, 2)
    out = _positional_lookup(table16, x, d)
    return out
